# baseline (speedup 1.0000x reference)
import jax
import jax.numpy as jnp
from jax import lax
from jax.experimental import pallas as pl
from jax.experimental.pallas import tpu as pltpu
from jax.experimental.pallas import tpu_sc as plsc

EMB = 256
L = 16
NCH = EMB // L
N_IDX = 200
N_POS = 20
N_NEG = 32
MARGIN_ = 0.1
EPS_ = 1e-8


def _zeros():
    return tuple(jnp.zeros((L,), jnp.float32) for _ in range(NCH))


def _sum_rows(rows_ref, lo, hi, init):
    def body(i, acc):
        return tuple(acc[c] + rows_ref[i, pl.ds(c * L, L)] for c in range(NCH))
    return lax.fori_loop(lo, hi, body, init)


def _allreduce_sum(x):
    lane = lax.iota(jnp.int32, L)
    for sh in (8, 4, 2, 1):
        x = x + x.at[lane ^ sh].get(mode="promise_in_bounds")
    return x


def _dot(u, v):
    acc = u[0] * v[0]
    for c in range(1, NCH):
        acc = acc + u[c] * v[c]
    return _allreduce_sum(acc)


def _rsqrt_vec(x):
    i = lax.bitcast_convert_type(x, jnp.int32)
    i = jnp.int32(0x5F3759DF) - lax.shift_right_logical(i, 1)
    y = lax.bitcast_convert_type(i, jnp.float32)
    for _ in range(4):
        y = y * (1.5 - 0.5 * x * y * y)
    return y


def _sqrt_vec(x):
    return jnp.where(x > 0.0, x * _rsqrt_vec(x), 0.0)


def _body(idx_hbm, tp_hbm, tn_hbm, word_hbm, tag_hbm, out_hbm,
          idxa_v, idxb_v, idxc_v, tp_v, tn_v,
          rows_a, rows_b, rows_c, rows_p, rows_n, res_v, sem):
    cid = lax.axis_index("c")
    sid = lax.axis_index("s")

    @pl.when(jnp.logical_and(cid == 0, sid == 0))
    def _():
        pltpu.sync_copy(idx_hbm.at[pl.ds(0, 128)], idxa_v)
        pltpu.sync_copy(idx_hbm.at[pl.ds(128, 64)], idxb_v)
        pltpu.sync_copy(idx_hbm.at[pl.ds(N_IDX - 16, 16)], idxc_v)
        pltpu.sync_copy(tp_hbm, tp_v.at[pl.ds(0, N_POS)])
        pltpu.sync_copy(tn_hbm, tn_v)
        tail = tp_v[pl.ds(16, 16)]
        lane = lax.iota(jnp.int32, L)
        tp_v[pl.ds(16, 16)] = tail.at[
            jnp.where(lane < N_POS - 16, lane, 0)
        ].get(mode="promise_in_bounds")
        h1 = pltpu.async_copy(word_hbm.at[idxa_v], rows_a, sem)
        h2 = pltpu.async_copy(word_hbm.at[idxb_v], rows_b, sem)
        h3 = pltpu.async_copy(word_hbm.at[idxc_v], rows_c, sem)
        h4 = pltpu.async_copy(tag_hbm.at[tp_v], rows_p, sem)
        h5 = pltpu.async_copy(tag_hbm.at[tn_v], rows_n, sem)
        h1.wait()
        h2.wait()
        h3.wait()
        h4.wait()
        h5.wait()

        xs = _sum_rows(rows_a, 0, 128, _zeros())
        xs = _sum_rows(rows_b, 0, 64, xs)
        xs = _sum_rows(rows_c, 192 - (N_IDX - 16), 16, xs)
        ys = _sum_rows(rows_p, 0, N_POS, _zeros())
        ng = _sum_rows(rows_n, 0, N_NEG, _zeros())

        dot_xn = _dot(xs, ng)
        dot_xy = _dot(xs, ys)
        nx2 = _dot(xs, xs)
        ny2 = _dot(ys, ys)
        nn2 = _dot(ng, ng)

        vnx = _sqrt_vec(nx2)
        vny = _sqrt_vec(ny2)
        vnn = _sqrt_vec(nn2)
        den_n = jnp.maximum(vnx * vnn, EPS_)
        den_y = jnp.maximum(vnx * vny, EPS_)
        crude = dot_xn / den_n - dot_xy / den_y + MARGIN_
        res_v[...] = jnp.maximum(crude, 0.0)
        pltpu.sync_copy(res_v, out_hbm)


def kernel(idx, targets_pos, targets_neg, word_embs, tag_embs):
    mesh = plsc.VectorSubcoreMesh(core_axis_name="c", subcore_axis_name="s")
    k = pl.kernel(
        _body,
        mesh=mesh,
        out_type=jax.ShapeDtypeStruct((L,), jnp.float32),
        scratch_types=[
            pltpu.VMEM((128,), jnp.int32),
            pltpu.VMEM((64,), jnp.int32),
            pltpu.VMEM((16,), jnp.int32),
            pltpu.VMEM((32,), jnp.int32),
            pltpu.VMEM((N_NEG,), jnp.int32),
            pltpu.VMEM((128, EMB), jnp.float32),
            pltpu.VMEM((64, EMB), jnp.float32),
            pltpu.VMEM((16, EMB), jnp.float32),
            pltpu.VMEM((32, EMB), jnp.float32),
            pltpu.VMEM((N_NEG, EMB), jnp.float32),
            pltpu.VMEM((L,), jnp.float32),
            pltpu.SemaphoreType.DMA,
        ],
    )
    out = k(idx, targets_pos, targets_neg, word_embs, tag_embs)
    return out[0]


# speedup vs baseline: 1.3351x; 1.3351x over previous
import jax
import jax.numpy as jnp
from jax import lax
from jax.experimental import pallas as pl
from jax.experimental.pallas import tpu as pltpu
from jax.experimental.pallas import tpu_sc as plsc

EMB = 256
L = 16
NCH = EMB // L
N_IDX = 200
N_POS = 20
N_NEG = 32
MARGIN_ = 0.1
EPS_ = 1e-8

N_WORD_TILES = 13
POS_TILE = 13
NEG_TILE0 = 14


def _zeros():
    return tuple(jnp.zeros((L,), jnp.float32) for _ in range(NCH))


def _sum_rows(rows_ref, lo, hi, init):
    def body(i, acc):
        return tuple(acc[c] + rows_ref[i, pl.ds(c * L, L)] for c in range(NCH))
    return lax.fori_loop(lo, hi, body, init)


def _allreduce_sum(x):
    lane = lax.iota(jnp.int32, L)
    for sh in (8, 4, 2, 1):
        x = x + x.at[lane ^ sh].get(mode="promise_in_bounds")
    return x


def _dot(u, v):
    acc = u[0] * v[0]
    for c in range(1, NCH):
        acc = acc + u[c] * v[c]
    return _allreduce_sum(acc)


def _rsqrt_vec(x):
    i = lax.bitcast_convert_type(x, jnp.int32)
    i = jnp.int32(0x5F3759DF) - lax.shift_right_logical(i, 1)
    y = lax.bitcast_convert_type(i, jnp.float32)
    for _ in range(4):
        y = y * (1.5 - 0.5 * x * y * y)
    return y


def _sqrt_vec(x):
    return jnp.where(x > 0.0, x * _rsqrt_vec(x), 0.0)


def _store_part(part_v, acc):
    for c in range(NCH):
        part_v[pl.ds(c * L, L)] = acc[c]


def _body(idx_hbm, tp_hbm, tn_hbm, word_hbm, tag_hbm, out_hbm,
          idx_v, rows_v, part_v, all_v, res_v, shared, sem):
    sid = lax.axis_index("s")

    @pl.when(sid < N_WORD_TILES)
    def _():
        base = pl.multiple_of(jnp.where(sid == 12, N_IDX - 16, L * sid), 8)
        pltpu.sync_copy(idx_hbm.at[pl.ds(base, 16)], idx_v.at[pl.ds(0, 16)])
        pltpu.async_copy(word_hbm.at[idx_v.at[pl.ds(0, 16)]],
                         rows_v.at[pl.ds(0, 16)], sem).wait()
        lo = jnp.where(sid == 12, 200 - 192, 0)
        acc = _sum_rows(rows_v, lo, 16, _zeros())
        _store_part(part_v, acc)
        pltpu.sync_copy(part_v, shared.at[sid])

    @pl.when(sid == POS_TILE)
    def _():
        pltpu.sync_copy(tp_hbm, idx_v.at[pl.ds(0, N_POS)])
        tail = idx_v[pl.ds(16, 16)]
        lane = lax.iota(jnp.int32, L)
        idx_v[pl.ds(16, 16)] = tail.at[
            jnp.where(lane < N_POS - 16, lane, 0)
        ].get(mode="promise_in_bounds")
        pltpu.async_copy(tag_hbm.at[idx_v], rows_v, sem).wait()
        acc = _sum_rows(rows_v, 0, N_POS, _zeros())
        _store_part(part_v, acc)
        pltpu.sync_copy(part_v, shared.at[sid])

    @pl.when(sid >= NEG_TILE0)
    def _():
        base = pl.multiple_of(16 * (sid - NEG_TILE0), 8)
        pltpu.sync_copy(tn_hbm.at[pl.ds(base, 16)], idx_v.at[pl.ds(0, 16)])
        pltpu.async_copy(tag_hbm.at[idx_v.at[pl.ds(0, 16)]],
                         rows_v.at[pl.ds(0, 16)], sem).wait()
        acc = _sum_rows(rows_v, 0, 16, _zeros())
        _store_part(part_v, acc)
        pltpu.sync_copy(part_v, shared.at[sid])

    plsc.subcore_barrier()

    @pl.when(sid == 0)
    def _():
        pltpu.sync_copy(shared, all_v)
        xs = _sum_rows(all_v, 0, N_WORD_TILES, _zeros())
        ys = tuple(all_v[POS_TILE, pl.ds(c * L, L)] for c in range(NCH))
        ng = tuple(all_v[NEG_TILE0, pl.ds(c * L, L)] +
                   all_v[NEG_TILE0 + 1, pl.ds(c * L, L)] for c in range(NCH))

        dot_xn = _dot(xs, ng)
        dot_xy = _dot(xs, ys)
        nx2 = _dot(xs, xs)
        ny2 = _dot(ys, ys)
        nn2 = _dot(ng, ng)

        vnx = _sqrt_vec(nx2)
        vny = _sqrt_vec(ny2)
        vnn = _sqrt_vec(nn2)
        den_n = jnp.maximum(vnx * vnn, EPS_)
        den_y = jnp.maximum(vnx * vny, EPS_)
        crude = dot_xn / den_n - dot_xy / den_y + MARGIN_
        res_v[...] = jnp.maximum(crude, 0.0)
        pltpu.sync_copy(res_v, out_hbm)


def kernel(idx, targets_pos, targets_neg, word_embs, tag_embs):
    mesh = plsc.VectorSubcoreMesh(core_axis_name="c", subcore_axis_name="s",
                                  num_cores=1)
    k = pl.kernel(
        _body,
        mesh=mesh,
        out_type=jax.ShapeDtypeStruct((L,), jnp.float32),
        scratch_types=[
            pltpu.VMEM((32,), jnp.int32),
            pltpu.VMEM((32, EMB), jnp.float32),
            pltpu.VMEM((EMB,), jnp.float32),
            pltpu.VMEM((16, EMB), jnp.float32),
            pltpu.VMEM((L,), jnp.float32),
            pltpu.VMEM_SHARED((16, EMB), jnp.float32),
            pltpu.SemaphoreType.DMA,
        ],
    )
    out = k(idx, targets_pos, targets_neg, word_embs, tag_embs)
    return out[0]
